# Initial kernel scaffold; baseline (speedup 1.0000x reference)
import jax
import jax.numpy as jnp
from jax.experimental import pallas as pl


def kernel(x, edge_index, batch_index, edge_attr, params):
    raise NotImplementedError("write your pallas kernel here")


# speedup vs baseline: 2.3095x; 2.3095x over previous
import functools

import jax
import jax.numpy as jnp
from jax import lax
from jax.experimental import pallas as pl
from jax.experimental.pallas import tpu as pltpu
from jax.experimental.pallas import tpu_sc as plsc

N_NODES = 10000
N_GROUPS = 64
F = 32
F2 = F * F
DE = 11
DEP = 16

NC = 2
NS = 16
NW = NC * NS
CS = 128

N_ACC = 10016
EB = 2048


def _mesh():
    return plsc.VectorSubcoreMesh(
        core_axis_name="c", subcore_axis_name="s", num_cores=NC, num_subcores=NS
    )


def _make_gather(e_pad):
    per_w = e_pad // NW
    n_ch = per_w // CS

    @functools.partial(
        pl.kernel,
        out_type=jax.ShapeDtypeStruct((e_pad, F), jnp.float32),
        mesh=_mesh(),
        scratch_types=[
            pltpu.VMEM((CS,), jnp.int32),
            pltpu.VMEM((CS, F), jnp.float32),
            pltpu.SemaphoreType.DMA,
        ],
        compiler_params=pltpu.CompilerParams(use_tc_tiling_on_sc=False),
    )
    def gather_k(table_hbm, idx_hbm, out_hbm, idx_v, rows_v, sem):
        wid = lax.axis_index("s") * NC + lax.axis_index("c")
        base = wid * per_w

        def body(c, carry):
            off = base + c * CS
            pltpu.sync_copy(idx_hbm.at[pl.ds(off, CS)], idx_v)
            pltpu.async_copy(table_hbm.at[idx_v], rows_v, sem).wait()
            pltpu.sync_copy(rows_v, out_hbm.at[pl.ds(off, CS)])
            return carry

        lax.fori_loop(0, n_ch, body, 0)

    return gather_k


def _make_scatter(e_pad):
    per_w = e_pad // NW
    n_ch = per_w // CS
    rpt = N_ACC // NS

    @functools.partial(
        pl.kernel,
        out_type=jax.ShapeDtypeStruct((NC, N_ACC, F), jnp.float32),
        mesh=_mesh(),
        scratch_types=[
            pltpu.VMEM((CS,), jnp.int32),
            pltpu.VMEM((CS, F), jnp.float32),
            pltpu.VMEM_SHARED((N_ACC, F), jnp.float32),
            pltpu.SemaphoreType.DMA,
        ],
        compiler_params=pltpu.CompilerParams(use_tc_tiling_on_sc=False),
    )
    def scatter_k(msg_hbm, dst_hbm, zeros_hbm, out_hbm, idx_v, msg_v, acc_sh, sem):
        cid = lax.axis_index("c")
        sid = lax.axis_index("s")
        wid = sid * NC + cid
        base = wid * per_w
        r0 = sid * rpt
        pltpu.sync_copy(zeros_hbm.at[pl.ds(r0, rpt)], acc_sh.at[pl.ds(r0, rpt)])
        plsc.subcore_barrier()

        def body(c, carry):
            off = base + c * CS
            pltpu.sync_copy(dst_hbm.at[pl.ds(off, CS)], idx_v)
            pltpu.sync_copy(msg_hbm.at[pl.ds(off, CS)], msg_v)
            pltpu.sync_copy(msg_v, acc_sh.at[idx_v], add=True)
            return carry

        lax.fori_loop(0, n_ch, body, 0)
        plsc.subcore_barrier()
        pltpu.sync_copy(acc_sh.at[pl.ds(r0, rpt)], out_hbm.at[cid, pl.ds(r0, rpt)])

    return scatter_k


def _leaky(v):
    return jnp.where(v > 0, v, 0.01 * v)


def _edge_body(ea_ref, xs_ref, a1_ref, b1_ref, a2p_ref, rep_ref, b2m_ref, msg_ref):
    ea = ea_ref[...]
    xs = xs_ref[...]
    h = _leaky(
        jax.lax.dot(ea, a1_ref[...], preferred_element_type=jnp.float32)
        + b1_ref[...]
    )
    y = jax.lax.dot(xs, a2p_ref[...], preferred_element_type=jnp.float32)
    hr = jax.lax.dot(h, rep_ref[...], preferred_element_type=jnp.float32)
    p = y * hr
    p = p[:, :512] + p[:, 512:]
    p = p[:, :256] + p[:, 256:]
    p = p[:, :128] + p[:, 128:]
    p = p[:, :64] + p[:, 64:]
    msg = p[:, :32] + p[:, 32:]
    msg = msg + jax.lax.dot(xs, b2m_ref[...], preferred_element_type=jnp.float32)
    msg_ref[...] = msg


def _make_edge(e_pad):
    grid = e_pad // EB
    return pl.pallas_call(
        _edge_body,
        grid=(grid,),
        in_specs=[
            pl.BlockSpec((EB, DEP), lambda i: (i, 0)),
            pl.BlockSpec((EB, F), lambda i: (i, 0)),
            pl.BlockSpec((DEP, F), lambda i: (0, 0)),
            pl.BlockSpec((1, F), lambda i: (0, 0)),
            pl.BlockSpec((F, F2), lambda i: (0, 0)),
            pl.BlockSpec((F, F2), lambda i: (0, 0)),
            pl.BlockSpec((F, F), lambda i: (0, 0)),
        ],
        out_specs=pl.BlockSpec((EB, F), lambda i: (i, 0)),
        out_shape=jax.ShapeDtypeStruct((e_pad, F), jnp.float32),
    )


def _node_bn_body(parts_ref, x_ref, root_ref, bias_ref, g_ref, b_ref, out_ref):
    agg = parts_ref[0, :N_NODES, :] + parts_ref[1, :N_NODES, :]
    v = agg + jax.lax.dot(x_ref[...], root_ref[...],
                          preferred_element_type=jnp.float32) + bias_ref[...]
    mu = jnp.mean(v, axis=0, keepdims=True)
    var = jnp.mean((v - mu) ** 2, axis=0, keepdims=True)
    hn = g_ref[...] * (v - mu) * lax.rsqrt(var + 1e-5) + b_ref[...]
    out_ref[...] = _leaky(hn)


def _node_body(parts_ref, x_ref, root_ref, bias_ref, out_ref):
    agg = parts_ref[0, :N_NODES, :] + parts_ref[1, :N_NODES, :]
    v = agg + jax.lax.dot(x_ref[...], root_ref[...],
                          preferred_element_type=jnp.float32) + bias_ref[...]
    out_ref[...] = _leaky(v)


_node_bn = pl.pallas_call(
    _node_bn_body,
    out_shape=jax.ShapeDtypeStruct((N_NODES, F), jnp.float32),
)

_node_plain = pl.pallas_call(
    _node_body,
    out_shape=jax.ShapeDtypeStruct((N_NODES, F), jnp.float32),
)


def _pool_body(h_ref, bi_ref, bir_ref, w_ref, b_ref, out_ref, mx_ref):
    h = h_ref[...]
    bi = bi_ref[...]
    onehot_t = (
        lax.broadcasted_iota(jnp.int32, (N_GROUPS, N_NODES), 0) == bir_ref[...]
    ).astype(jnp.float32)
    h1 = jnp.concatenate([h, jnp.ones((N_NODES, F), jnp.float32)], axis=1)
    sums1 = jax.lax.dot(onehot_t, h1, preferred_element_type=jnp.float32)
    sums = sums1[:, :F]
    cnt = sums1[:, F:F + 1]
    meanp = sums / jnp.maximum(cnt, 1.0)

    def body(g, carry):
        m = bi == g
        mx_ref[pl.ds(g, 1), :] = jnp.max(
            jnp.where(m, h, -jnp.inf), axis=0, keepdims=True
        )
        return carry

    lax.fori_loop(0, N_GROUPS, body, 0)
    pooled = jnp.concatenate([mx_ref[...], meanp], axis=1)
    out_ref[...] = (
        jnp.sum(pooled * w_ref[...], axis=1, keepdims=True) + b_ref[...]
    )


_pool = pl.pallas_call(
    _pool_body,
    out_shape=jax.ShapeDtypeStruct((N_GROUPS, 1), jnp.float32),
    scratch_shapes=[pltpu.VMEM((N_GROUPS, F), jnp.float32)],
)


def kernel(x, edge_index, batch_index, edge_attr, params):
    e = edge_index.shape[1]
    e_pad = -(-e // (NW * CS)) * (NW * CS)

    src = edge_index[0].astype(jnp.int32)
    dst = edge_index[1].astype(jnp.int32)
    src_p = jnp.concatenate([src, jnp.zeros((e_pad - e,), jnp.int32)])
    dst_p = jnp.concatenate([dst, jnp.full((e_pad - e,), N_NODES, jnp.int32)])
    ea_p = jnp.pad(edge_attr, ((0, e_pad - e), (0, DEP - DE)))
    zeros_acc = jnp.zeros((N_ACC, F), jnp.float32)
    rep = jnp.kron(jnp.eye(F, dtype=jnp.float32), jnp.ones((1, F), jnp.float32))

    gather_k = _make_gather(e_pad)
    scatter_k = _make_scatter(e_pad)
    edge_k = _make_edge(e_pad)

    h = x
    layers = [("conv0", "bn0"), ("conv1", "bn1"), ("conv2", None), ("conv3", None)]
    for conv, bn in layers:
        p = params[conv]
        a1p = jnp.pad(p["A1"], ((0, DEP - DE), (0, 0)))
        a2p = p["A2"].reshape(F, F, F).transpose(1, 0, 2).reshape(F, F2)
        b2m = p["b2"].reshape(F, F)

        xs = gather_k(h, src_p)
        msg = edge_k(ea_p, xs, a1p, p["b1"][None, :], a2p, rep, b2m)
        parts = scatter_k(msg, dst_p, zeros_acc)
        if bn is not None:
            q = params[bn]
            h = _node_bn(parts, h, p["root"], p["bias"][None, :],
                         q["gamma"][None, :], q["beta"][None, :])
        else:
            h = _node_plain(parts, h, p["root"], p["bias"][None, :])

    bi = batch_index.astype(jnp.int32).reshape(N_NODES, 1)
    bi_row = batch_index.astype(jnp.int32).reshape(1, N_NODES)
    w_row = params["fc"]["W"].reshape(1, 2 * F)
    b_fc = params["fc"]["b"].reshape(1, 1)
    return _pool(h, bi, bi_row, w_row, b_fc)


# speedup vs baseline: 2.6134x; 1.1316x over previous
import functools

import jax
import jax.numpy as jnp
from jax import lax
from jax.experimental import pallas as pl
from jax.experimental.pallas import tpu as pltpu
from jax.experimental.pallas import tpu_sc as plsc

N_NODES = 10000
N_GROUPS = 64
F = 32
F2 = F * F
DE = 11
DEP = 16

NC = 2
NS = 16
NW = NC * NS
CS = 128

N_ACC = 10016
EB = 2048

BF = jnp.bfloat16
F32 = jnp.float32


def _mesh():
    return plsc.VectorSubcoreMesh(
        core_axis_name="c", subcore_axis_name="s", num_cores=NC, num_subcores=NS
    )


def _make_gather(e_pad):
    per_w = e_pad // NW
    n_ch = per_w // CS

    @functools.partial(
        pl.kernel,
        out_type=jax.ShapeDtypeStruct((NW * n_ch, CS, F), F32),
        mesh=_mesh(),
        scratch_types=[
            pltpu.VMEM((n_ch, CS), jnp.int32),
            pltpu.VMEM((n_ch, CS, F), F32),
            pltpu.SemaphoreType.DMA,
        ],
        compiler_params=pltpu.CompilerParams(use_tc_tiling_on_sc=False),
    )
    def gather_k(table_hbm, idx_hbm, out_hbm, idx_v, rows_v, sem):
        wid = lax.axis_index("s") * NC + lax.axis_index("c")
        pltpu.sync_copy(idx_hbm.at[pl.ds(wid * n_ch, n_ch)], idx_v)

        def fire(c, carry):
            pltpu.async_copy(table_hbm.at[idx_v.at[c]], rows_v.at[c], sem)
            return carry

        lax.fori_loop(0, n_ch, fire, 0)
        pltpu.make_async_copy(
            out_hbm.at[pl.ds(wid * n_ch, n_ch)], rows_v, sem
        ).wait()
        pltpu.sync_copy(rows_v, out_hbm.at[pl.ds(wid * n_ch, n_ch)])

    return gather_k


def _make_scatter(e_pad):
    per_w = e_pad // NW
    n_ch = per_w // CS
    rpt = N_ACC // NS

    @functools.partial(
        pl.kernel,
        out_type=jax.ShapeDtypeStruct((NC, N_ACC, F), F32),
        mesh=_mesh(),
        scratch_types=[
            pltpu.VMEM((n_ch, CS), jnp.int32),
            pltpu.VMEM((n_ch, CS, F), F32),
            pltpu.VMEM_SHARED((N_ACC, F), F32),
            pltpu.SemaphoreType.DMA,
        ],
        compiler_params=pltpu.CompilerParams(use_tc_tiling_on_sc=False),
    )
    def scatter_k(msg_hbm, dst_hbm, zeros_hbm, out_hbm, idx_v, msg_v, acc_sh, sem):
        cid = lax.axis_index("c")
        sid = lax.axis_index("s")
        wid = sid * NC + cid
        r0 = sid * rpt
        pltpu.sync_copy(zeros_hbm.at[pl.ds(r0, rpt)], acc_sh.at[pl.ds(r0, rpt)])
        pltpu.sync_copy(dst_hbm.at[pl.ds(wid * n_ch, n_ch)], idx_v)
        pltpu.sync_copy(msg_hbm.at[pl.ds(wid * n_ch, n_ch)], msg_v)
        plsc.subcore_barrier()

        def fire(c, carry):
            pltpu.async_copy(msg_v.at[c], acc_sh.at[idx_v.at[c]], sem, add=True)
            return carry

        lax.fori_loop(0, n_ch, fire, 0)

        def drain(c, carry):
            pltpu.make_async_copy(msg_v.at[c], acc_sh.at[idx_v.at[c]], sem).wait()
            return carry

        lax.fori_loop(0, n_ch, drain, 0)
        plsc.subcore_barrier()
        pltpu.sync_copy(acc_sh.at[pl.ds(r0, rpt)], out_hbm.at[cid, pl.ds(r0, rpt)])

    return scatter_k


def _leaky(v):
    return jnp.where(v > 0, v, 0.01 * v)


def _edge_body(ea_ref, xs_ref, a1_ref, b1_ref, a2_ref, rep_ref, b2_ref, msg_ref):
    ea16 = ea_ref[...]
    xs16 = xs_ref[...].astype(BF)
    h = _leaky(
        jax.lax.dot(ea16, a1_ref[...], preferred_element_type=F32)
        + b1_ref[...]
    )
    w = jax.lax.dot(h.astype(BF), a2_ref[...], preferred_element_type=F32)
    w16 = (w + b2_ref[...]).astype(BF).astype(F32)
    xr = jax.lax.dot(xs16, rep_ref[...], preferred_element_type=F32)
    p = xr * w16
    p = p[:, :512] + p[:, 512:]
    p = p[:, :256] + p[:, 256:]
    p = p[:, :128] + p[:, 128:]
    p = p[:, :64] + p[:, 64:]
    msg_ref[...] = p[:, :32] + p[:, 32:]


def _make_edge(e_pad):
    grid = e_pad // EB
    return pl.pallas_call(
        _edge_body,
        grid=(grid,),
        in_specs=[
            pl.BlockSpec((EB, DEP), lambda i: (i, 0)),
            pl.BlockSpec((EB, F), lambda i: (i, 0)),
            pl.BlockSpec((DEP, F), lambda i: (0, 0)),
            pl.BlockSpec((1, F), lambda i: (0, 0)),
            pl.BlockSpec((F, F2), lambda i: (0, 0)),
            pl.BlockSpec((F, F2), lambda i: (0, 0)),
            pl.BlockSpec((1, F2), lambda i: (0, 0)),
        ],
        out_specs=pl.BlockSpec((EB, F), lambda i: (i, 0)),
        out_shape=jax.ShapeDtypeStruct((e_pad, F), F32),
    )


def _root_term(x_ref, root_ref):
    return jax.lax.dot(
        x_ref[...].astype(BF), root_ref[...], preferred_element_type=F32
    )


def _node_bn_body(parts_ref, x_ref, root_ref, bias_ref, g_ref, b_ref, out_ref):
    agg = parts_ref[0, :N_NODES, :] + parts_ref[1, :N_NODES, :]
    v = agg + _root_term(x_ref, root_ref) + bias_ref[...]
    mu = jnp.mean(v, axis=0, keepdims=True)
    var = jnp.mean((v - mu) ** 2, axis=0, keepdims=True)
    hn = g_ref[...] * (v - mu) / jnp.sqrt(var + 1e-5) + b_ref[...]
    out_ref[...] = _leaky(hn)


def _node_body(parts_ref, x_ref, root_ref, bias_ref, out_ref):
    agg = parts_ref[0, :N_NODES, :] + parts_ref[1, :N_NODES, :]
    v = agg + _root_term(x_ref, root_ref) + bias_ref[...]
    out_ref[...] = _leaky(v)


_node_bn = pl.pallas_call(
    _node_bn_body,
    out_shape=jax.ShapeDtypeStruct((N_NODES, F), F32),
)

_node_plain = pl.pallas_call(
    _node_body,
    out_shape=jax.ShapeDtypeStruct((N_NODES, F), F32),
)


def _pool_body(h_ref, bi_ref, bir_ref, out_ref, mx_ref):
    h = h_ref[...]
    bi = bi_ref[...]
    onehot_t = (
        lax.broadcasted_iota(jnp.int32, (N_GROUPS, N_NODES), 0) == bir_ref[...]
    ).astype(F32)
    h1 = jnp.concatenate([h, jnp.ones((N_NODES, F), F32)], axis=1)
    sums1 = jax.lax.dot(
        onehot_t, h1,
        precision=jax.lax.Precision.HIGHEST,
        preferred_element_type=F32,
    )
    sums = sums1[:, :F]
    cnt = sums1[:, F:F + 1]
    meanp = sums / jnp.maximum(cnt, 1.0)

    def body(g, carry):
        m = bi == g
        mx_ref[pl.ds(g, 1), :] = jnp.max(
            jnp.where(m, h, -jnp.inf), axis=0, keepdims=True
        )
        return carry

    lax.fori_loop(0, N_GROUPS, body, 0)
    out_ref[...] = jnp.concatenate([mx_ref[...], meanp], axis=1)


_pool = pl.pallas_call(
    _pool_body,
    out_shape=jax.ShapeDtypeStruct((N_GROUPS, 2 * F), F32),
    scratch_shapes=[pltpu.VMEM((N_GROUPS, F), F32)],
)


def kernel(x, edge_index, batch_index, edge_attr, params):
    e = edge_index.shape[1]
    e_pad = -(-e // (NW * CS)) * (NW * CS)
    n_ch = e_pad // NW // CS

    src = edge_index[0].astype(jnp.int32)
    dst = edge_index[1].astype(jnp.int32)
    src_p = jnp.concatenate(
        [src, jnp.zeros((e_pad - e,), jnp.int32)]).reshape(NW * n_ch, CS)
    dst_p = jnp.concatenate(
        [dst, jnp.full((e_pad - e,), N_NODES, jnp.int32)]).reshape(NW * n_ch, CS)
    ea16 = jnp.pad(edge_attr, ((0, e_pad - e), (0, DEP - DE))).astype(BF)
    zeros_acc = jnp.zeros((N_ACC, F), F32)
    rep16 = jnp.kron(jnp.eye(F, dtype=F32), jnp.ones((1, F), F32)).astype(BF)

    gather_k = _make_gather(e_pad)
    scatter_k = _make_scatter(e_pad)
    edge_k = _make_edge(e_pad)

    h = x
    layers = [("conv0", "bn0"), ("conv1", "bn1"), ("conv2", None), ("conv3", None)]
    for conv, bn in layers:
        p = params[conv]
        a1p16 = jnp.pad(p["A1"], ((0, DEP - DE), (0, 0))).astype(BF)

        xs = gather_k(h, src_p).reshape(e_pad, F)
        msg = edge_k(ea16, xs, a1p16, p["b1"][None, :], p["A2"].astype(BF),
                     rep16, p["b2"][None, :])
        parts = scatter_k(msg.reshape(NW * n_ch, CS, F), dst_p, zeros_acc)
        if bn is not None:
            q = params[bn]
            h = _node_bn(parts, h, p["root"].astype(BF), p["bias"][None, :],
                         q["gamma"][None, :], q["beta"][None, :])
        else:
            h = _node_plain(parts, h, p["root"].astype(BF), p["bias"][None, :])

    bi = batch_index.astype(jnp.int32).reshape(N_NODES, 1)
    bi_row = batch_index.astype(jnp.int32).reshape(1, N_NODES)
    pooled = _pool(h, bi, bi_row)
    return pooled @ params["fc"]["W"] + params["fc"]["b"]
